# baseline (speedup 1.0000x reference)
import dataclasses
import functools

import numpy as np
import jax
import jax.numpy as jnp
from jax import lax
from jax.experimental import pallas as pl
from jax.experimental.pallas import tpu as pltpu
from jax.experimental.pallas import tpu_sc as plsc

_SHUFFLE_CHANNEL = 2048
_TOTAL = 8192
_NC, _NS, _L = 2, 16, 16
_NW = _NC * _NS
_ROWS = 4 * 1024
_RPW = _ROWS // _NW
_RBLK = 2
_NBLK = _RPW // _RBLK
_NBUF = 4
_BLK_EL = _RBLK * _TOTAL
_M = _RBLK * _SHUFFLE_CHANNEL


def _build_index() -> np.ndarray:
    pkey = jax.random.key(42)
    random_sort = jax.random.permutation(pkey, _TOTAL)[:_SHUFFLE_CHANNEL]
    random_index = jnp.sort(random_sort)
    rs = np.asarray(random_sort).astype(np.int32)
    ri = np.asarray(random_index).astype(np.int32)
    gsrc = np.concatenate([rs + r * _TOTAL for r in range(_RBLK)])
    sdst = np.concatenate([ri + r * _TOTAL for r in range(_RBLK)])
    return np.concatenate([gsrc, sdst])


_IDX = _build_index()


@jax.jit
def _sc_shuffle(x1d, idx):
    mesh = plsc.VectorSubcoreMesh(
        core_axis_name="c", subcore_axis_name="s",
        num_cores=_NC, num_subcores=_NS,
    )

    cp = pltpu.CompilerParams()
    if "needs_layout_passes" in pltpu.CompilerParams.__dataclass_fields__:
        cp = dataclasses.replace(cp, needs_layout_passes=False)

    @functools.partial(
        pl.kernel,
        mesh=mesh,
        compiler_params=cp,
        out_type=jax.ShapeDtypeStruct((_ROWS * _TOTAL,), jnp.float32),
        scratch_types=(
            [pltpu.VMEM((2 * _M,), jnp.int32)]
            + [pltpu.VMEM((_BLK_EL,), jnp.float32) for _ in range(_NBUF)]
            + [pltpu.VMEM((_M,), jnp.float32)]
            + [pltpu.SemaphoreType.DMA for _ in range(2 * _NBUF)]
        ),
    )
    def k(x_hbm, idx_hbm, o_hbm, idx_v, b0, b1, b2, b3, g_v,
          si0, si1, si2, si3, so0, so1, so2, so3):
        bufs = (b0, b1, b2, b3)
        sins = (si0, si1, si2, si3)
        souts = (so0, so1, so2, so3)
        wid = lax.axis_index("s") * _NC + lax.axis_index("c")
        wbase = wid * _RPW * _TOTAL
        pltpu.sync_copy(idx_hbm, idx_v)

        def in_src(b):
            return x_hbm.at[pl.ds(wbase + b * _BLK_EL, _BLK_EL)]

        def out_dst(b):
            return o_hbm.at[pl.ds(wbase + b * _BLK_EL, _BLK_EL)]

        def fix(buf):
            @plsc.parallel_loop(0, _M, step=_L, unroll=8)
            def _gather(j):
                g_v[pl.ds(j, _L)] = plsc.load_gather(buf, [idx_v[pl.ds(j, _L)]])

            @plsc.parallel_loop(0, _M, step=_L, unroll=8)
            def _scatter(j):
                plsc.store_scatter(buf, [idx_v[pl.ds(_M + j, _L)]],
                                   g_v[pl.ds(j, _L)])

        pltpu.async_copy(in_src(0), bufs[0], sins[0])
        pltpu.async_copy(in_src(1), bufs[1], sins[1])

        @pl.loop(0, _NBLK, step=_NBUF)
        def _grp(g):
            for kk in range(_NBUF):
                b = g + kk
                buf, sin, sout = bufs[kk], sins[kk], souts[kk]
                kn = (kk + 2) % _NBUF
                bufn, sinn, soutn = bufs[kn], sins[kn], souts[kn]

                pltpu.make_async_copy(in_src(b), buf, sin).wait()
                fix(buf)
                pltpu.async_copy(buf, out_dst(b), sout)

                @pl.when(b >= 2)
                def _retire():
                    pltpu.make_async_copy(bufn, out_dst(b - 2), soutn).wait()

                @pl.when(b + 2 < _NBLK)
                def _prefetch():
                    pltpu.async_copy(in_src(b + 2), bufn, sinn)

        for bb in (_NBLK - 2, _NBLK - 1):
            kk = bb % _NBUF
            pltpu.make_async_copy(bufs[kk], out_dst(bb), souts[kk]).wait()

    return k(x1d, idx)


def kernel(input):
    x1d = input.reshape(_ROWS * _TOTAL)
    out = _sc_shuffle(x1d, jnp.asarray(_IDX))
    return out.reshape(input.shape)


# speedup vs baseline: 3.0818x; 3.0818x over previous
import dataclasses
import functools

import numpy as np
import jax
import jax.numpy as jnp
from jax import lax
from jax.experimental import pallas as pl
from jax.experimental.pallas import tpu as pltpu
from jax.experimental.pallas import tpu_sc as plsc

_SHUFFLE_CHANNEL = 2048
_TOTAL = 8192
_NC, _NS, _L = 2, 16, 16
_NW = _NC * _NS
_ROWS = 4 * 1024
_RPW = _ROWS // _NW
_NBUF = 8
_PREF = 4


def _build_index() -> np.ndarray:
    pkey = jax.random.key(42)
    random_sort = jax.random.permutation(pkey, _TOTAL)[:_SHUFFLE_CHANNEL]
    random_index = jnp.sort(random_sort)
    rs = np.asarray(random_sort).astype(np.int32)
    ri = np.asarray(random_index).astype(np.int32)
    return np.concatenate([rs, ri])


_IDX = _build_index()


@jax.jit
def _sc_shuffle(x2d, idx):
    mesh = plsc.VectorSubcoreMesh(
        core_axis_name="c", subcore_axis_name="s",
        num_cores=_NC, num_subcores=_NS,
    )

    cp = pltpu.CompilerParams()
    if "needs_layout_passes" in pltpu.CompilerParams.__dataclass_fields__:
        cp = dataclasses.replace(cp, needs_layout_passes=False)

    @functools.partial(
        pl.kernel,
        mesh=mesh,
        compiler_params=cp,
        out_type=jax.ShapeDtypeStruct((_ROWS, _TOTAL), jnp.float32),
        scratch_types=(
            [pltpu.VMEM((2 * _SHUFFLE_CHANNEL,), jnp.int32)]
            + [pltpu.VMEM((_TOTAL,), jnp.float32) for _ in range(_NBUF)]
            + [pltpu.VMEM((_SHUFFLE_CHANNEL,), jnp.float32)]
            + [pltpu.SemaphoreType.DMA for _ in range(2 * _NBUF)]
        ),
    )
    def k(x_hbm, idx_hbm, o_hbm, idx_v, *rest):
        bufs = rest[:_NBUF]
        g_v = rest[_NBUF]
        sins = rest[_NBUF + 1:2 * _NBUF + 1]
        souts = rest[2 * _NBUF + 1:]
        wid = lax.axis_index("s") * _NC + lax.axis_index("c")
        base = wid * _RPW
        pltpu.sync_copy(idx_hbm, idx_v)

        def fix(buf):
            @plsc.parallel_loop(0, _SHUFFLE_CHANNEL, step=_L, unroll=8)
            def _gather(j):
                g_v[pl.ds(j, _L)] = plsc.load_gather(buf, [idx_v[pl.ds(j, _L)]])

            @plsc.parallel_loop(0, _SHUFFLE_CHANNEL, step=_L, unroll=8)
            def _scatter(j):
                plsc.store_scatter(buf, [idx_v[pl.ds(_SHUFFLE_CHANNEL + j, _L)]],
                                   g_v[pl.ds(j, _L)])

        for p in range(_PREF):
            pltpu.async_copy(x_hbm.at[base + p], bufs[p], sins[p])

        @pl.loop(0, _RPW, step=_NBUF)
        def _grp(g):
            for kk in range(_NBUF):
                b = g + kk
                kn = (kk + _PREF) % _NBUF

                pltpu.make_async_copy(x_hbm.at[base + b], bufs[kk], sins[kk]).wait()
                fix(bufs[kk])
                pltpu.async_copy(bufs[kk], o_hbm.at[base + b], souts[kk])

                @pl.when(b >= _NBUF - _PREF)
                def _retire():
                    pltpu.make_async_copy(
                        bufs[kn], o_hbm.at[base + b - (_NBUF - _PREF)],
                        souts[kn]).wait()

                @pl.when(b + _PREF < _RPW)
                def _prefetch():
                    pltpu.async_copy(x_hbm.at[base + b + _PREF], bufs[kn],
                                     sins[kn])

        for bb in range(_RPW - (_NBUF - _PREF), _RPW):
            kk = bb % _NBUF
            pltpu.make_async_copy(bufs[kk], o_hbm.at[base + bb],
                                  souts[kk]).wait()

    return k(x2d, idx)


def kernel(input):
    x2d = input.reshape(_ROWS, _TOTAL)
    out = _sc_shuffle(x2d, jnp.asarray(_IDX))
    return out.reshape(input.shape)
